# baseline (speedup 1.0000x reference)
import functools

import jax
import jax.numpy as jnp
import numpy as np
from jax import lax
from jax.experimental import pallas as pl
from jax.experimental.pallas import tpu as pltpu
from jax.experimental.pallas import tpu_sc as plsc

C = 128
K = 16
KK = K + 1
N = 10000
NW = 32
NS = 16
N_PAD = 10240
NBLK = 2048
NB = N_PAD // NBLK
SLOW_C = 0
PF = 512
PS = 128
FTOT = NS * PF
OSUB = 16


def _tc_tables_body(y_ref, w_ref, b_ref, z_ref):
    z = jax.lax.dot_general(
        y_ref[...], w_ref[0],
        (((0,), (0,)), ((), ())),
        preferred_element_type=jnp.float32,
    )
    z_ref[0] = z + b_ref[0]


def _build_tables(y2, wall, ball):
    return pl.pallas_call(
        _tc_tables_body,
        grid=(NB, KK),
        in_specs=[
            pl.BlockSpec((C, NBLK), lambda nb, g: (0, nb)),
            pl.BlockSpec((1, C, C), lambda nb, g: (g, 0, 0)),
            pl.BlockSpec((1, 1, C), lambda nb, g: (g, 0, 0)),
        ],
        out_specs=pl.BlockSpec((1, NBLK, C), lambda nb, g: (g, nb, 0)),
        out_shape=jax.ShapeDtypeStruct((KK, N_PAD, C), jnp.float32),
    )(y2, wall, ball)


def _segs(total):
    out, o = [], 0
    while o < total:
        s = min(128, total - o)
        out.append((o, s))
        o += s
    return tuple(out)


@functools.partial(
    pl.kernel,
    out_type=jax.ShapeDtypeStruct((N, C), jnp.float32),
    mesh=plsc.VectorSubcoreMesh(core_axis_name="c", subcore_axis_name="s"),
    scratch_types=[
        pltpu.VMEM((K, PF), jnp.int32),
        pltpu.VMEM((PF, C), jnp.float32),
        pltpu.SemaphoreType.DMA,
        pltpu.SemaphoreType.DMA,
    ],
)
def _sc_gather_sum(ztab, idxw, out, idxt_v, acc_v, gsem, bsem):
    cc = lax.axis_index("c")
    sid = lax.axis_index("s")
    wid = sid * 2 + cc
    pltpu.async_copy(idxw.at[wid], idxt_v, bsem)

    def run(start, count, guarded):
        pltpu.async_copy(ztab.at[pl.ds(K * N_PAD + start, count)],
                         acc_v.at[pl.ds(0, count)], gsem)
        pltpu.make_async_copy(idxw.at[wid], idxt_v, bsem).wait()
        pltpu.make_async_copy(ztab.at[pl.ds(K * N_PAD + start, count)],
                              acc_v.at[pl.ds(0, count)], gsem).wait()
        for g in range(K):
            for o, s in _segs(count):
                pltpu.async_copy(ztab.at[idxt_v.at[g, pl.ds(o, s)]],
                                 acc_v.at[pl.ds(o, s)], gsem, add=True)
        for g in range(K):
            for o, s in _segs(count):
                pltpu.make_async_copy(ztab.at[idxt_v.at[g, pl.ds(o, s)]],
                                      acc_v.at[pl.ds(o, s)], gsem).wait()
        if not guarded:
            pltpu.sync_copy(acc_v.at[pl.ds(0, count)],
                            out.at[pl.ds(start, count)])
        else:
            for j in range(count // OSUB):
                @pl.when(start + (j + 1) * OSUB <= N)
                def _(j=j):
                    pltpu.sync_copy(acc_v.at[pl.ds(j * OSUB, OSUB)],
                                    out.at[pl.ds(start + j * OSUB, OSUB)])

    @pl.when(cc != SLOW_C)
    def _():
        run(sid * PF, PF, guarded=False)

    @pl.when(cc == SLOW_C)
    def _():
        run(FTOT + sid * PS, PS, guarded=True)


def _worker_starts():
    starts = np.zeros(NW, np.int64)
    for s in range(NS):
        for c in range(2):
            w = s * 2 + c
            starts[w] = FTOT + s * PS if c == SLOW_C else s * PF
    return starts


def kernel(x, y, y_xyz, params, idx):
    p = params
    y2 = y[0]
    wv3 = p['Wv'].reshape(C, C, K)
    a = jnp.transpose(wv3, (2, 1, 0))
    wall = jnp.concatenate([a, -a.sum(axis=0, keepdims=True)], axis=0)
    ball = jnp.zeros((KK, 1, C), jnp.float32).at[K, 0].set(p['bv'])

    idx2 = idx[0].astype(jnp.int32)
    idxf = jnp.zeros((K, N_PAD), jnp.int32).at[:, :N].set(
        idx2.T + (jnp.arange(K, dtype=jnp.int32) * N_PAD)[:, None])
    cols = _worker_starts()[:, None] + np.arange(PF)[None, :]
    cols = np.minimum(cols, N_PAD - 1)
    idxw = jnp.transpose(idxf[:, cols], (1, 0, 2))

    zall = _build_tables(y2, wall, ball)
    ztab = zall.reshape(KK * N_PAD, C)

    out_rows = _sc_gather_sum(ztab, idxw)
    return out_rows.T[None]


# speedup vs baseline: 1.3452x; 1.3452x over previous
import functools

import jax
import jax.numpy as jnp
from jax import lax
from jax.experimental import pallas as pl
from jax.experimental.pallas import tpu as pltpu
from jax.experimental.pallas import tpu_sc as plsc

C = 128
K = 16
KK = K + 1
N = 10000
NW = 32
N_PAD = 10240
PW = N_PAD // NW
NBLK = 2048
NB = N_PAD // NBLK
SEGS = ((0, 128), (128, 128), (256, 64))
OSUB = 80


def _tc_tables_body(y_ref, w_ref, b_ref, z_ref):
    z = jax.lax.dot_general(
        y_ref[...], w_ref[0],
        (((0,), (0,)), ((), ())),
        preferred_element_type=jnp.float32,
    )
    z_ref[0] = z + b_ref[0]


def _build_tables(y2, wall, ball):
    return pl.pallas_call(
        _tc_tables_body,
        grid=(NB, KK),
        in_specs=[
            pl.BlockSpec((C, NBLK), lambda nb, g: (0, nb)),
            pl.BlockSpec((1, C, C), lambda nb, g: (g, 0, 0)),
            pl.BlockSpec((1, 1, C), lambda nb, g: (g, 0, 0)),
        ],
        out_specs=pl.BlockSpec((1, NBLK, C), lambda nb, g: (g, nb, 0)),
        out_shape=jax.ShapeDtypeStruct((KK, N_PAD, C), jnp.float32),
    )(y2, wall, ball)


@functools.partial(
    pl.kernel,
    out_type=jax.ShapeDtypeStruct((N, C), jnp.float32),
    mesh=plsc.VectorSubcoreMesh(core_axis_name="c", subcore_axis_name="s"),
    scratch_types=[
        pltpu.VMEM((K, PW), jnp.int32),
        pltpu.VMEM((PW, C), jnp.float32),
        pltpu.SemaphoreType.DMA,
        pltpu.SemaphoreType.DMA,
    ],
)
def _sc_gather_sum(ztab, idxw, out, idxt_v, acc_v, gsem, bsem):
    wid = lax.axis_index("s") * 2 + lax.axis_index("c")
    base_pt = wid * PW
    pltpu.async_copy(idxw.at[wid], idxt_v, bsem)
    pltpu.async_copy(ztab.at[pl.ds(K * N_PAD + base_pt, PW)], acc_v, gsem)
    pltpu.make_async_copy(idxw.at[wid], idxt_v, bsem).wait()
    pltpu.make_async_copy(ztab.at[pl.ds(K * N_PAD + base_pt, PW)],
                          acc_v, gsem).wait()
    for g in range(K):
        for o, s in SEGS:
            pltpu.async_copy(ztab.at[idxt_v.at[g, pl.ds(o, s)]],
                             acc_v.at[pl.ds(o, s)], gsem, add=True)
    for g in range(K):
        for o, s in SEGS:
            pltpu.make_async_copy(ztab.at[idxt_v.at[g, pl.ds(o, s)]],
                                  acc_v.at[pl.ds(o, s)], gsem).wait()
    for j in range(PW // OSUB):
        @pl.when(base_pt + (j + 1) * OSUB <= N)
        def _(j=j):
            pltpu.sync_copy(acc_v.at[pl.ds(j * OSUB, OSUB)],
                            out.at[pl.ds(base_pt + j * OSUB, OSUB)])


def kernel(x, y, y_xyz, params, idx):
    p = params
    y2 = y[0]
    wv3 = p['Wv'].reshape(C, C, K)
    a = jnp.transpose(wv3, (2, 1, 0))
    wall = jnp.concatenate([a, -a.sum(axis=0, keepdims=True)], axis=0)
    ball = jnp.zeros((KK, 1, C), jnp.float32).at[K, 0].set(p['bv'])

    idx2 = idx[0].astype(jnp.int32)
    idxp = jnp.zeros((N_PAD, K), jnp.int32).at[:N].set(idx2)
    idxw = (jnp.transpose(idxp.reshape(NW, PW, K), (0, 2, 1))
            + (jnp.arange(K, dtype=jnp.int32) * N_PAD)[None, :, None])

    zall = _build_tables(y2, wall, ball)
    ztab = zall.reshape(KK * N_PAD, C)

    out_rows = _sc_gather_sum(ztab, idxw)
    return out_rows.T[None]


# speedup vs baseline: 1.8817x; 1.3988x over previous
import functools

import jax
import jax.numpy as jnp
from jax import lax
from jax.experimental import pallas as pl
from jax.experimental.pallas import tpu as pltpu
from jax.experimental.pallas import tpu_sc as plsc

C = 128
K = 16
KK = K + 1
N = 10000
NW = 32
NS = 16
N_PAD = 10240
NBLK = 2048
NB = N_PAD // NBLK
SLOW_C = 0
PF = 512
PS = 128
FTOT = NS * PF
OSUB = 16


def _tc_tables_body(y_ref, w_ref, b_ref, z_ref):
    z = jax.lax.dot_general(
        y_ref[...], w_ref[0],
        (((0,), (0,)), ((), ())),
        preferred_element_type=jnp.float32,
    )
    z_ref[0] = z + b_ref[0]


def _build_tables(y2, wall, ball):
    return pl.pallas_call(
        _tc_tables_body,
        grid=(NB, KK),
        in_specs=[
            pl.BlockSpec((C, NBLK), lambda nb, g: (0, nb)),
            pl.BlockSpec((1, C, C), lambda nb, g: (g, 0, 0)),
            pl.BlockSpec((1, 1, C), lambda nb, g: (g, 0, 0)),
        ],
        out_specs=pl.BlockSpec((1, NBLK, C), lambda nb, g: (g, nb, 0)),
        out_shape=jax.ShapeDtypeStruct((KK, N_PAD, C), jnp.float32),
    )(y2, wall, ball)


def _segs(total):
    out, o = [], 0
    while o < total:
        s = min(128, total - o)
        out.append((o, s))
        o += s
    return tuple(out)


@functools.partial(
    pl.kernel,
    out_type=jax.ShapeDtypeStruct((N, C), jnp.float32),
    mesh=plsc.VectorSubcoreMesh(core_axis_name="c", subcore_axis_name="s"),
    scratch_types=[
        pltpu.VMEM((K, PF), jnp.int32),
        pltpu.VMEM((PF, C), jnp.float32),
        pltpu.SemaphoreType.DMA,
        pltpu.SemaphoreType.DMA,
    ],
)
def _sc_gather_sum(ztab, idxw, out, idxt_v, acc_v, gsem, bsem):
    cc = lax.axis_index("c")
    sid = lax.axis_index("s")
    row = jnp.where(cc != SLOW_C, sid, NS + sid)
    pltpu.async_copy(idxw.at[row], idxt_v, bsem)

    def run(start, count, guarded):
        pltpu.async_copy(ztab.at[pl.ds(K * N_PAD + start, count)],
                         acc_v.at[pl.ds(0, count)], gsem)
        pltpu.make_async_copy(idxw.at[row], idxt_v, bsem).wait()
        pltpu.make_async_copy(ztab.at[pl.ds(K * N_PAD + start, count)],
                              acc_v.at[pl.ds(0, count)], gsem).wait()
        for g in range(K):
            for o, s in _segs(count):
                pltpu.async_copy(ztab.at[idxt_v.at[g, pl.ds(o, s)]],
                                 acc_v.at[pl.ds(o, s)], gsem, add=True)
        for g in range(K):
            for o, s in _segs(count):
                pltpu.make_async_copy(ztab.at[idxt_v.at[g, pl.ds(o, s)]],
                                      acc_v.at[pl.ds(o, s)], gsem).wait()
        if not guarded:
            pltpu.sync_copy(acc_v.at[pl.ds(0, count)],
                            out.at[pl.ds(start, count)])
        else:
            for j in range(count // OSUB):
                @pl.when(start + (j + 1) * OSUB <= N)
                def _(j=j):
                    pltpu.sync_copy(acc_v.at[pl.ds(j * OSUB, OSUB)],
                                    out.at[pl.ds(start + j * OSUB, OSUB)])

    @pl.when(cc != SLOW_C)
    def _():
        run(sid * PF, PF, guarded=False)

    @pl.when(cc == SLOW_C)
    def _():
        run(FTOT + sid * PS, PS, guarded=True)


def kernel(x, y, y_xyz, params, idx):
    p = params
    y2 = y[0]
    wv3 = p['Wv'].reshape(C, C, K)
    a = jnp.transpose(wv3, (2, 1, 0))
    wall = jnp.concatenate([a, -a.sum(axis=0, keepdims=True)], axis=0)
    ball = jnp.zeros((KK, 1, C), jnp.float32).at[K, 0].set(p['bv'])

    idx2 = idx[0].astype(jnp.int32)
    idxp = jnp.zeros((N_PAD, K), jnp.int32).at[:N].set(idx2)
    offs = (jnp.arange(K, dtype=jnp.int32) * N_PAD)[None, :, None]
    fast = jnp.transpose(idxp[:FTOT].reshape(NS, PF, K), (0, 2, 1)) + offs
    slow = jnp.transpose(idxp[FTOT:].reshape(NS, PS, K), (0, 2, 1)) + offs
    slow = jnp.pad(slow, ((0, 0), (0, 0), (0, PF - PS)))
    idxw = jnp.concatenate([fast, slow], axis=0)

    zall = _build_tables(y2, wall, ball)
    ztab = zall.reshape(KK * N_PAD, C)

    out_rows = _sc_gather_sum(ztab, idxw)
    return out_rows.T[None]


# speedup vs baseline: 2.0253x; 1.0763x over previous
import functools

import jax
import jax.numpy as jnp
from jax import lax
from jax.experimental import pallas as pl
from jax.experimental.pallas import tpu as pltpu
from jax.experimental.pallas import tpu_sc as plsc

C = 128
K = 16
KK = K + 1
N = 10000
NW = 32
NS = 16
N_PAD = 10240
NBLK = 2048
NB = N_PAD // NBLK
SLOW_C = 0
PF = 576
PS = 64
FTOT = NS * PF
OSUB = 16


def _tc_tables_body(y_ref, w_ref, b_ref, z_ref):
    z = jax.lax.dot_general(
        y_ref[...], w_ref[0],
        (((0,), (0,)), ((), ())),
        preferred_element_type=jnp.float32,
    )
    z_ref[0] = z + b_ref[0]


def _build_tables(y2, wall, ball):
    return pl.pallas_call(
        _tc_tables_body,
        grid=(NB, KK),
        in_specs=[
            pl.BlockSpec((C, NBLK), lambda nb, g: (0, nb)),
            pl.BlockSpec((1, C, C), lambda nb, g: (g, 0, 0)),
            pl.BlockSpec((1, 1, C), lambda nb, g: (g, 0, 0)),
        ],
        out_specs=pl.BlockSpec((1, NBLK, C), lambda nb, g: (g, nb, 0)),
        out_shape=jax.ShapeDtypeStruct((KK, N_PAD, C), jnp.float32),
    )(y2, wall, ball)


def _segs(total):
    out, o = [], 0
    while o < total:
        s = min(128, total - o)
        out.append((o, s))
        o += s
    return tuple(out)


@functools.partial(
    pl.kernel,
    out_type=jax.ShapeDtypeStruct((N, C), jnp.float32),
    mesh=plsc.VectorSubcoreMesh(core_axis_name="c", subcore_axis_name="s"),
    scratch_types=[
        pltpu.VMEM((K, PF), jnp.int32),
        pltpu.VMEM((K, 128), jnp.int32),
        pltpu.VMEM((PF, C), jnp.float32),
        pltpu.SemaphoreType.DMA,
        pltpu.SemaphoreType.DMA,
    ],
)
def _sc_gather_sum(ztab, idxw_f, idxw_s, out, idxt_v, idxs_v, acc_v, gsem, bsem):
    cc = lax.axis_index("c")
    sid = lax.axis_index("s")

    def stage_idx(count):
        if count == PF:
            pltpu.async_copy(idxw_f.at[sid], idxt_v, bsem)
            return pltpu.make_async_copy(idxw_f.at[sid], idxt_v, bsem)
        pltpu.async_copy(idxw_s.at[sid], idxs_v, bsem)
        return pltpu.make_async_copy(idxw_s.at[sid], idxs_v, bsem)

    def run(start, count, guarded):
        iref = idxt_v if count == PF else idxs_v
        idx_cp = stage_idx(count)
        pltpu.async_copy(ztab.at[pl.ds(K * N_PAD + start, count)],
                         acc_v.at[pl.ds(0, count)], gsem)
        idx_cp.wait()
        pltpu.make_async_copy(ztab.at[pl.ds(K * N_PAD + start, count)],
                              acc_v.at[pl.ds(0, count)], gsem).wait()
        for g in range(K):
            for o, s in _segs(count):
                pltpu.async_copy(ztab.at[iref.at[g, pl.ds(o, s)]],
                                 acc_v.at[pl.ds(o, s)], gsem, add=True)
        for g in range(K):
            for o, s in _segs(count):
                pltpu.make_async_copy(ztab.at[iref.at[g, pl.ds(o, s)]],
                                      acc_v.at[pl.ds(o, s)], gsem).wait()
        if not guarded:
            pltpu.sync_copy(acc_v.at[pl.ds(0, count)],
                            out.at[pl.ds(start, count)])
        else:
            for j in range(count // OSUB):
                @pl.when(start + (j + 1) * OSUB <= N)
                def _(j=j):
                    pltpu.sync_copy(acc_v.at[pl.ds(j * OSUB, OSUB)],
                                    out.at[pl.ds(start + j * OSUB, OSUB)])

    @pl.when(cc != SLOW_C)
    def _():
        run(sid * PF, PF, guarded=False)

    @pl.when(cc == SLOW_C)
    def _():
        run(FTOT + sid * PS, PS, guarded=True)


def kernel(x, y, y_xyz, params, idx):
    p = params
    y2 = y[0]
    wv3 = p['Wv'].reshape(C, C, K)
    a = jnp.transpose(wv3, (2, 1, 0))
    wall = jnp.concatenate([a, -a.sum(axis=0, keepdims=True)], axis=0)
    ball = jnp.zeros((KK, 1, C), jnp.float32).at[K, 0].set(p['bv'])

    idx2 = idx[0].astype(jnp.int32)
    idxp = jnp.zeros((N_PAD, K), jnp.int32).at[:N].set(idx2)
    offs = (jnp.arange(K, dtype=jnp.int32) * N_PAD)[None, :, None]
    idxw_f = jnp.transpose(idxp[:FTOT].reshape(NS, PF, K), (0, 2, 1)) + offs
    idxw_s = jnp.transpose(idxp[FTOT:].reshape(NS, PS, K), (0, 2, 1)) + offs
    idxw_s = jnp.pad(idxw_s, ((0, 0), (0, 0), (0, 128 - PS)))

    zall = _build_tables(y2, wall, ball)
    ztab = zall.reshape(KK * N_PAD, C)

    out_rows = _sc_gather_sum(ztab, idxw_f, idxw_s)
    return out_rows.T[None]
